# baseline (speedup 1.0000x reference)
import jax
import jax.numpy as jnp
from jax.experimental import pallas as pl
from jax.experimental.pallas import tpu as pltpu

_B, _N = 4096, 50
_MAX_OUT = 4
_D = 64
_NUM_EMB = 8
_BB = 128
_ROWS = _BB * _N
_HALF = _MAX_OUT * _D // 2


def _emb_kernel(subs_ref, mask_ref, tab_ref, buf_ref, cnt_ref):
    j = pl.program_id(1)
    mf = mask_ref[...].astype(jnp.float32)

    @pl.when(j == 0)
    def _():
        subs = subs_ref[...]
        ids = jnp.clip(subs + 1, 0, _NUM_EMB - 1)
        k_iota = jax.lax.broadcasted_iota(jnp.int32, (1, 1, _NUM_EMB), 2)
        oh = (ids[..., None] == k_iota).astype(jnp.float32) * mf[..., None]
        prim = jax.lax.dot_general(
            oh.reshape(_ROWS, _NUM_EMB), tab_ref[...],
            (((1,), (0,)), ((), ())), preferred_element_type=jnp.float32)
        buf_ref[:, :, 0:_D] = prim.reshape(_BB, _N, _D)
        buf_ref[:, :, _D:] = jnp.zeros((_BB, _N, _HALF - _D), jnp.float32)

    @pl.when(j == 1)
    def _():
        buf_ref[...] = jnp.zeros((_BB, _N, _HALF), jnp.float32)

    cnt_ref[...] = mf


def kernel(subs, mask, embed_table):
    mask_i = mask.astype(jnp.int32)
    grid = (_B // _BB, 2)
    buf, cnt = pl.pallas_call(
        _emb_kernel,
        grid=grid,
        in_specs=[
            pl.BlockSpec((_BB, _N), lambda i, j: (i, 0)),
            pl.BlockSpec((_BB, _N), lambda i, j: (i, 0)),
            pl.BlockSpec((_NUM_EMB, _D), lambda i, j: (0, 0)),
        ],
        out_specs=[
            pl.BlockSpec((_BB, _N, _HALF), lambda i, j: (i, 0, j)),
            pl.BlockSpec((_BB, _N), lambda i, j: (i, 0)),
        ],
        out_shape=[
            jax.ShapeDtypeStruct((_B, _N, _MAX_OUT * _D), jnp.float32),
            jax.ShapeDtypeStruct((_B, _N), jnp.float32),
        ],
        compiler_params=pltpu.CompilerParams(
            dimension_semantics=("parallel", "arbitrary")),
    )(subs, mask_i, embed_table)
    return buf.reshape(_B, _N, _MAX_OUT, _D), cnt


# speedup vs baseline: 1.2729x; 1.2729x over previous
import jax
import jax.numpy as jnp
from jax.experimental import pallas as pl
from jax.experimental.pallas import tpu as pltpu

_B, _N = 4096, 50
_MAX_OUT = 4
_D = 64
_NUM_EMB = 8
_BB = 128
_ROWS = _BB * _N


def _emb_kernel(subs_ref, mask_ref, tab_ref, buf_ref, cnt_ref):
    subs = subs_ref[...]
    mi = mask_ref[...]
    ids = jnp.clip(subs + 1, 0, _NUM_EMB - 1)
    eff = jnp.where(mi > 0, ids, -1)
    k_iota = jax.lax.broadcasted_iota(jnp.int32, (1, 1, _NUM_EMB), 2)
    oh = (eff[..., None] == k_iota).astype(jnp.float32)
    prim = jax.lax.dot_general(
        oh.reshape(_ROWS, _NUM_EMB), tab_ref[...],
        (((1,), (0,)), ((), ())), preferred_element_type=jnp.float32)
    buf_ref[:, :, 0:_D] = prim.reshape(_BB, _N, _D)
    buf_ref[:, :, _D:] = jnp.zeros((_BB, _N, (_MAX_OUT - 1) * _D), jnp.float32)
    cnt_ref[...] = mi.astype(jnp.float32)


def kernel(subs, mask, embed_table):
    mask_i = mask.astype(jnp.int32)
    grid = (_B // _BB,)
    buf, cnt = pl.pallas_call(
        _emb_kernel,
        grid=grid,
        in_specs=[
            pl.BlockSpec((_BB, _N), lambda i: (i, 0)),
            pl.BlockSpec((_BB, _N), lambda i: (i, 0)),
            pl.BlockSpec((_NUM_EMB, _D), lambda i: (0, 0)),
        ],
        out_specs=[
            pl.BlockSpec((_BB, _N, _MAX_OUT * _D), lambda i: (i, 0, 0)),
            pl.BlockSpec((_BB, _N), lambda i: (i, 0)),
        ],
        out_shape=[
            jax.ShapeDtypeStruct((_B, _N, _MAX_OUT * _D), jnp.float32),
            jax.ShapeDtypeStruct((_B, _N), jnp.float32),
        ],
        compiler_params=pltpu.CompilerParams(
            dimension_semantics=("parallel",)),
    )(subs, mask_i, embed_table)
    return buf.reshape(_B, _N, _MAX_OUT, _D), cnt
